# Initial kernel scaffold; baseline (speedup 1.0000x reference)
import jax
import jax.numpy as jnp
from jax.experimental import pallas as pl


def kernel(node_feature, W):
    raise NotImplementedError("write your pallas kernel here")


# speedup vs baseline: 11.9880x; 11.9880x over previous
import functools

import jax
import jax.numpy as jnp
from jax import lax
from jax.experimental import pallas as pl
from jax.experimental.pallas import tpu as pltpu
from jax.experimental.pallas import tpu_sc as plsc

B, H, F, D, V = 4096, 50, 26, 32, 100000
NC, NS, L = 2, 16, 16
NW = NC * NS
POS = B * H
PPW = POS // NW
P = 64
NCHUNK = PPW // P
RPC = P * F
NSUB = RPC // 128
JROWS = PPW * F // 128

_mesh = plsc.VectorSubcoreMesh(
    core_axis_name="c", subcore_axis_name="s", num_cores=NC, num_subcores=NS
)


@functools.partial(
    pl.kernel,
    out_type=jax.ShapeDtypeStruct((POS * D // 128, 128), jnp.float32),
    mesh=_mesh,
    scratch_types=[
        pltpu.VMEM((RPC,), jnp.int32),
        pltpu.VMEM((NSUB, 128), jnp.int32),
        pltpu.VMEM((NSUB, 128), jnp.int32),
        pltpu.VMEM((RPC, D), jnp.float32),
        pltpu.VMEM((P * D // 128, 128), jnp.float32),
        pltpu.SemaphoreType.DMA,
    ],
    compiler_params=pltpu.CompilerParams(use_tc_tiling_on_sc=False),
)
def _embed_sum(nf_hbm, w_hbm, out_hbm, raw_v, off_v, fidx_v, rows_v, outb_v, sem):
    cid = lax.axis_index("c")
    sid = lax.axis_index("s")
    wid = sid * NC + cid

    def off_body(j, carry):
        for r in range(8):
            e = j * 128 + r * 16 + lax.iota(jnp.int32, 16)
            off_v[j, pl.ds(r * 16, 16)] = (e % F) * V
        return carry

    lax.fori_loop(0, NSUB, off_body, 0)

    def chunk_body(c, carry):
        pos_base = wid * PPW + c * P
        ebase = pl.multiple_of(pos_base * F, 128)
        pltpu.sync_copy(nf_hbm.at[pl.ds(ebase, RPC)], raw_v)

        def fidx_body(j, carry):
            for r in range(8):
                fidx_v[j, pl.ds(r * 16, 16)] = (
                    raw_v[pl.ds(j * 128 + r * 16, 16)] + off_v[j, pl.ds(r * 16, 16)]
                )
            return carry

        lax.fori_loop(0, NSUB, fidx_body, 0)

        cps = [
            pltpu.async_copy(
                w_hbm.at[fidx_v.at[j]], rows_v.at[pl.ds(j * 128, 128)], sem
            )
            for j in range(NSUB)
        ]
        for cp in cps:
            cp.wait()

        def acc_body(q, carry):
            for k in range(4):
                base = (q * 4 + k) * F
                a0 = rows_v[base, pl.ds(0, 16)]
                a1 = rows_v[base, pl.ds(16, 16)]
                for i in range(1, F):
                    a0 = a0 + rows_v[base + i, pl.ds(0, 16)]
                    a1 = a1 + rows_v[base + i, pl.ds(16, 16)]
                outb_v[q, pl.ds(k * 32, 16)] = a0
                outb_v[q, pl.ds(k * 32 + 16, 16)] = a1
            return carry

        lax.fori_loop(0, P * D // 128, acc_body, 0)
        orow = pl.multiple_of(pos_base * D // 128, 16)
        pltpu.sync_copy(outb_v, out_hbm.at[pl.ds(orow, P * D // 128)])
        return carry

    lax.fori_loop(0, NCHUNK, chunk_body, 0)


def kernel(node_feature, W):
    nf1 = node_feature.reshape(POS * F)
    w2 = W.reshape(F * V, D)
    out = _embed_sum(nf1, w2)
    return out.reshape(B, H, D)
